# Initial kernel scaffold; baseline (speedup 1.0000x reference)
import jax
import jax.numpy as jnp
from jax.experimental import pallas as pl


def kernel(xp):
    raise NotImplementedError("write your pallas kernel here")


# speedup vs baseline: 1.8025x; 1.8025x over previous
import numpy as np
import jax
import jax.numpy as jnp
from jax.experimental import pallas as pl
from jax.experimental.pallas import tpu as pltpu

W_BINS = 8
H_BINS = 6


def _edges(size, n_bins):
    p = np.float32(size) / np.float32(n_bins)
    return [int(np.round(np.float32(i) * p)) for i in range(n_bins + 1)]


def _make_body(r_edges, c_edges, regular_rows, row_w):
    def body(x_ref, o_ref):
        x = x_ref[0]
        if regular_rows:
            xr = x.reshape(W_BINS, row_w, x.shape[1], x.shape[2])
            rm = jnp.max(xr, axis=1)
        else:
            rm = jnp.concatenate(
                [jnp.max(x[r_edges[i]:r_edges[i + 1]], axis=0, keepdims=True)
                 for i in range(W_BINS)], axis=0)
        cols = [jnp.max(rm[:, c_edges[j]:c_edges[j + 1], :], axis=1, keepdims=True)
                for j in range(H_BINS)]
        o_ref[0] = jnp.concatenate(cols, axis=1)
    return body


def kernel(xp):
    B, w, h, M = xp.shape
    wr, hr = max(w, W_BINS), max(h, H_BINS)
    if (wr, hr) != (w, h):
        xp = jax.image.resize(xp, (B, wr, hr, M), method="bilinear",
                              antialias=False)
    r_edges = _edges(wr, W_BINS)
    c_edges = _edges(hr, H_BINS)
    widths = [r_edges[i + 1] - r_edges[i] for i in range(W_BINS)]
    regular_rows = len(set(widths)) == 1
    row_w = widths[0]

    c_tile = M if M <= 256 else 256
    grid = (B, M // c_tile)
    out = pl.pallas_call(
        _make_body(r_edges, c_edges, regular_rows, row_w),
        grid=grid,
        in_specs=[pl.BlockSpec((1, wr, hr, c_tile), lambda b, c: (b, 0, 0, c))],
        out_specs=pl.BlockSpec((1, W_BINS, H_BINS, c_tile),
                               lambda b, c: (b, 0, 0, c)),
        out_shape=jax.ShapeDtypeStruct((B, W_BINS, H_BINS, M), xp.dtype),
        compiler_params=pltpu.CompilerParams(
            dimension_semantics=("parallel", "parallel")),
    )(xp)
    return out.reshape(B, W_BINS * H_BINS * M)


# speedup vs baseline: 1.9075x; 1.0583x over previous
import numpy as np
import jax
import jax.numpy as jnp
from jax.experimental import pallas as pl
from jax.experimental.pallas import tpu as pltpu

W_BINS = 8
H_BINS = 6


def _edges(size, n_bins):
    p = np.float32(size) / np.float32(n_bins)
    return [int(np.round(np.float32(i) * p)) for i in range(n_bins + 1)]


def _make_body(r_edges, c_edges, regular_rows, row_w):
    def body(x_ref, o_ref):
        x = x_ref[...]
        bt, _, hh, cc = x.shape
        if regular_rows:
            xr = x.reshape(bt, W_BINS, row_w, hh, cc)
            rm = jnp.max(xr, axis=2)
        else:
            rm = jnp.concatenate(
                [jnp.max(x[:, r_edges[i]:r_edges[i + 1]], axis=1, keepdims=True)
                 for i in range(W_BINS)], axis=1)
        cols = [jnp.max(rm[:, :, c_edges[j]:c_edges[j + 1], :], axis=2,
                        keepdims=True)
                for j in range(H_BINS)]
        o_ref[...] = jnp.concatenate(cols, axis=2)
    return body


def kernel(xp):
    B, w, h, M = xp.shape
    wr, hr = max(w, W_BINS), max(h, H_BINS)
    if (wr, hr) != (w, h):
        xp = jax.image.resize(xp, (B, wr, hr, M), method="bilinear",
                              antialias=False)
    r_edges = _edges(wr, W_BINS)
    c_edges = _edges(hr, H_BINS)
    widths = [r_edges[i + 1] - r_edges[i] for i in range(W_BINS)]
    regular_rows = len(set(widths)) == 1
    row_w = widths[0]

    b_tile = 2 if B % 2 == 0 else 1
    c_tile = M if M <= 512 else 512
    grid = (B // b_tile, M // c_tile)
    out = pl.pallas_call(
        _make_body(r_edges, c_edges, regular_rows, row_w),
        grid=grid,
        in_specs=[pl.BlockSpec((b_tile, wr, hr, c_tile),
                               lambda b, c: (b, 0, 0, c))],
        out_specs=pl.BlockSpec((b_tile, W_BINS, H_BINS, c_tile),
                               lambda b, c: (b, 0, 0, c)),
        out_shape=jax.ShapeDtypeStruct((B, W_BINS, H_BINS, M), xp.dtype),
        compiler_params=pltpu.CompilerParams(
            dimension_semantics=("parallel", "parallel")),
    )(xp)
    return out.reshape(B, W_BINS * H_BINS * M)


# speedup vs baseline: 1.9550x; 1.0249x over previous
import numpy as np
import jax
import jax.numpy as jnp
from jax.experimental import pallas as pl
from jax.experimental.pallas import tpu as pltpu

W_BINS = 8
H_BINS = 6


def _edges(size, n_bins):
    p = np.float32(size) / np.float32(n_bins)
    return [int(np.round(np.float32(i) * p)) for i in range(n_bins + 1)]


def _make_body(r_edges, c_edges, regular_rows, row_w):
    def body(x_ref, o_ref):
        x = x_ref[...]
        bt, _, hh, cc = x.shape
        if regular_rows:
            xr = x.reshape(bt, W_BINS, row_w, hh, cc)
            rm = jnp.max(xr, axis=2)
        else:
            rm = jnp.concatenate(
                [jnp.max(x[:, r_edges[i]:r_edges[i + 1]], axis=1, keepdims=True)
                 for i in range(W_BINS)], axis=1)
        cols = [jnp.max(rm[:, :, c_edges[j]:c_edges[j + 1], :], axis=2,
                        keepdims=True)
                for j in range(H_BINS)]
        o_ref[...] = jnp.concatenate(cols, axis=2)
    return body


def kernel(xp):
    B, w, h, M = xp.shape
    wr, hr = max(w, W_BINS), max(h, H_BINS)
    if (wr, hr) != (w, h):
        xp = jax.image.resize(xp, (B, wr, hr, M), method="bilinear",
                              antialias=False)
    r_edges = _edges(wr, W_BINS)
    c_edges = _edges(hr, H_BINS)
    widths = [r_edges[i + 1] - r_edges[i] for i in range(W_BINS)]
    regular_rows = len(set(widths)) == 1
    row_w = widths[0]

    b_tile = 1
    c_tile = M if M <= 512 else 512
    grid = (B // b_tile, M // c_tile)
    out = pl.pallas_call(
        _make_body(r_edges, c_edges, regular_rows, row_w),
        grid=grid,
        in_specs=[pl.BlockSpec((b_tile, wr, hr, c_tile),
                               lambda b, c: (b, 0, 0, c))],
        out_specs=pl.BlockSpec((b_tile, W_BINS, H_BINS, c_tile),
                               lambda b, c: (b, 0, 0, c)),
        out_shape=jax.ShapeDtypeStruct((B, W_BINS, H_BINS, M), xp.dtype),
        compiler_params=pltpu.CompilerParams(
            dimension_semantics=("parallel", "parallel")),
    )(xp)
    return out.reshape(B, W_BINS * H_BINS * M)
